# baseline (speedup 1.0000x reference)
import functools

import jax
import jax.numpy as jnp
from jax import lax
from jax.experimental import pallas as pl
from jax.experimental.pallas import tpu as pltpu
from jax.experimental.pallas import tpu_sc as plsc

_N = 10000
_E = 320000
_D = 128
_R_CUT = 5.0


def _lin1_body(x_ref, wT_ref, o_ref):
    o_ref[...] = jnp.dot(x_ref[...], wT_ref[...],
                         preferred_element_type=jnp.float32)


def _lin1(x, wT):
    return pl.pallas_call(
        _lin1_body,
        out_shape=jax.ShapeDtypeStruct((_N, _D), jnp.float32),
    )(x, wT)


_BE = 1280


def _col128(row):
    rid = lax.broadcasted_iota(jnp.int32, (128, 128), 0)
    lid = lax.broadcasted_iota(jnp.int32, (128, 128), 1)
    b = jnp.broadcast_to(row, (128, 128))
    return jnp.sum(jnp.where(rid == lid, b, 0.0), axis=1, keepdims=True)


def _filter_body(attrs_ref, ew_ref, wf1T_ref, bf1_ref, wf2T_ref, w_ref):
    a = attrs_ref[...].astype(jnp.bfloat16)
    h = jnp.dot(a, wf1T_ref[...], preferred_element_type=jnp.float32)
    h = h + bf1_ref[...]
    h = h * jax.nn.sigmoid(h)
    w = jnp.dot(h.astype(jnp.bfloat16), wf2T_ref[...],
                preferred_element_type=jnp.float32)
    ew = ew_ref[0]
    c = 0.5 * (jnp.cos(jnp.pi * ew / _R_CUT) + 1.0)
    c = c * (ew < _R_CUT).astype(jnp.float32)
    ccol = jnp.concatenate(
        [_col128(c[r:r + 1, :]) for r in range(_BE // 128)], axis=0)
    w_ref[...] = w * ccol


def _edge_filter(edge_attrs, edge_weights, wf1T, bf1, wf2T):
    ne = edge_attrs.shape[0]
    ge = ne // _BE
    ew2 = edge_weights.reshape(ge, _BE // 128, 128)
    return pl.pallas_call(
        _filter_body,
        grid=(ge,),
        in_specs=[
            pl.BlockSpec((_BE, _D), lambda i: (i, 0)),
            pl.BlockSpec((1, _BE // 128, 128), lambda i: (i, 0, 0)),
            pl.BlockSpec((_D, _D), lambda i: (0, 0)),
            pl.BlockSpec((1, _D), lambda i: (0, 0)),
            pl.BlockSpec((_D, _D), lambda i: (0, 0)),
        ],
        out_specs=pl.BlockSpec((_BE, _D), lambda i: (i, 0)),
        out_shape=jax.ShapeDtypeStruct((ne, _D), jnp.float32),
    )(edge_attrs, ew2, wf1T, bf1.reshape(1, _D), wf2T)


_NC, _NS, _L = 2, 16, 16
_NW = _NC * _NS
_NP = 10112
_RPT = _NP // _NS

_sc_mesh = plsc.VectorSubcoreMesh(core_axis_name="c", subcore_axis_name="s")


def _make_sc_scatter(perw, ch, nch):
    assert ch * nch == perw and nch % 2 == 1 and ch % 8 == 0 and ch <= 128

    @functools.partial(
        pl.kernel,
        out_type=jax.ShapeDtypeStruct((_NC, _NP, _D), jnp.float32),
        mesh=_sc_mesh,
        scratch_types=[
            pltpu.VMEM((perw,), jnp.int32),
            pltpu.VMEM((ch,), jnp.int32),
            pltpu.VMEM((ch,), jnp.int32),
            pltpu.VMEM((ch,), jnp.int32),
            pltpu.VMEM((ch,), jnp.int32),
            pltpu.VMEM((ch, _D), jnp.float32),
            pltpu.VMEM((ch, _D), jnp.float32),
            pltpu.VMEM((ch, _D), jnp.float32),
            pltpu.VMEM((ch, _D), jnp.float32),
            pltpu.VMEM_SHARED((_NP, _D), jnp.float32),
            pltpu.SemaphoreType.DMA,
            pltpu.SemaphoreType.DMA,
            pltpu.SemaphoreType.DMA,
            pltpu.SemaphoreType.DMA,
            pltpu.SemaphoreType.DMA,
            pltpu.SemaphoreType.DMA,
        ],
    )
    def sc_scatter(w_hbm, s_hbm, pk_hbm, z_hbm, out_hbm,
                   pk, srcb0, srcb1, dstb0, dstb1,
                   w0, w1, v0, v1, acc,
                   semw0, semw1, semg0, semg1, sems0, sems1):
        cid = lax.axis_index("c")
        sid = lax.axis_index("s")
        wid = cid * _NS + sid
        pltpu.sync_copy(z_hbm, acc.at[pl.ds(sid * _RPT, _RPT)])
        pltpu.sync_copy(pk_hbm.at[wid], pk)
        plsc.subcore_barrier()

        mask = jnp.full((_L,), 0xFFFF, jnp.int32)

        offs = list(range(0, ch - _L + 1, _L))
        if offs[-1] != ch - _L:
            offs.append(ch - _L)

        def unpack_idx(j, srcb, dstb):
            for o in offs:
                x = pk[pl.ds(j * ch + o, _L)]
                srcb[pl.ds(o, _L)] = x & mask
                dstb[pl.ds(o, _L)] = lax.shift_right_logical(x, 16)

        def start_loads(j, w_v, v_v, srcb, dstb, semw, semg):
            unpack_idx(j, srcb, dstb)
            base = wid * perw + j * ch
            pltpu.async_copy(w_hbm.at[pl.ds(base, ch)], w_v, semw)
            pltpu.async_copy(s_hbm.at[srcb], v_v, semg)

        def wait_loads(j, w_v, v_v, srcb, semw, semg):
            base = wid * perw + j * ch
            pltpu.make_async_copy(w_hbm.at[pl.ds(base, ch)], w_v, semw).wait()
            pltpu.make_async_copy(s_hbm.at[srcb], v_v, semg).wait()

        def multiply(w_v, v_v):
            def row(r, c2):
                for c in range(_D // _L):
                    sl = pl.ds(c * _L, _L)
                    v_v[r, sl] = v_v[r, sl] * w_v[r, sl]
                return c2
            lax.fori_loop(0, ch, row, 0)

        def start_scatter(v_v, dstb, sems):
            pltpu.async_copy(v_v, acc.at[dstb], sems, add=True)

        def wait_scatter(v_v, dstb, sems):
            pltpu.make_async_copy(v_v, acc.at[dstb], sems).wait()

        start_loads(0, w0, v0, srcb0, dstb0, semw0, semg0)

        def pair(h, carry):
            ja = 2 * h
            jb = ja + 1

            @pl.when(h > 0)
            def _():
                wait_scatter(v1, dstb1, sems1)
            start_loads(jb, w1, v1, srcb1, dstb1, semw1, semg1)
            wait_loads(ja, w0, v0, srcb0, semw0, semg0)
            multiply(w0, v0)
            start_scatter(v0, dstb0, sems0)

            wait_scatter(v0, dstb0, sems0)
            start_loads(ja + 2, w0, v0, srcb0, dstb0, semw0, semg0)
            wait_loads(jb, w1, v1, srcb1, semw1, semg1)
            multiply(w1, v1)
            start_scatter(v1, dstb1, sems1)
            return carry

        lax.fori_loop(0, nch // 2, pair, 0)
        wait_scatter(v1, dstb1, sems1)
        wait_loads(nch - 1, w0, v0, srcb0, semw0, semg0)
        multiply(w0, v0)
        start_scatter(v0, dstb0, sems0)
        wait_scatter(v0, dstb0, sems0)

        plsc.subcore_barrier()
        pltpu.sync_copy(acc.at[pl.ds(sid * _RPT, _RPT)],
                        out_hbm.at[cid, pl.ds(sid * _RPT, _RPT)])

    return sc_scatter


_ES = _E // 2
_SC_SLAB = _make_sc_scatter(_ES // _NW, 40, 125)


_BN = 1000
_GN = _N // _BN


def _node_body(pa_ref, pb_ref, recv_ref, w2T_ref, b2_ref, g_ref, be_ref,
               w3T_ref, b3_ref, o_ref):
    upd = pa_ref[0] + pa_ref[1] + pb_ref[0] + pb_ref[1]
    y = jnp.dot(upd, w2T_ref[...], preferred_element_type=jnp.float32)
    y = y + b2_ref[...]
    mu = jnp.mean(y, axis=-1, keepdims=True)
    yc = y - mu
    var = jnp.mean(yc * yc, axis=-1, keepdims=True)
    y = yc * lax.rsqrt(var + 1e-5) * g_ref[...] + be_ref[...]
    y = y * jax.nn.sigmoid(y)
    o_ref[...] = (jnp.dot(y, w3T_ref[...], preferred_element_type=jnp.float32)
                  + b3_ref[...] + recv_ref[...])


def _node_mlp(pa, pb, recv, w2T, b2, gamma, beta, w3T, b3):
    return pl.pallas_call(
        _node_body,
        grid=(_GN,),
        in_specs=[
            pl.BlockSpec((_NC, _BN, _D), lambda i: (0, i, 0)),
            pl.BlockSpec((_NC, _BN, _D), lambda i: (0, i, 0)),
            pl.BlockSpec((_BN, _D), lambda i: (i, 0)),
            pl.BlockSpec((_D, _D), lambda i: (0, 0)),
            pl.BlockSpec((1, _D), lambda i: (0, 0)),
            pl.BlockSpec((1, _D), lambda i: (0, 0)),
            pl.BlockSpec((1, _D), lambda i: (0, 0)),
            pl.BlockSpec((_D, _D), lambda i: (0, 0)),
            pl.BlockSpec((1, _D), lambda i: (0, 0)),
        ],
        out_specs=pl.BlockSpec((_BN, _D), lambda i: (i, 0)),
        out_shape=jax.ShapeDtypeStruct((_N, _D), jnp.float32),
    )(pa, pb, recv, w2T, b2.reshape(1, _D), gamma.reshape(1, _D),
      beta.reshape(1, _D), w3T, b3.reshape(1, _D))


def kernel(senders, receivers, edge_indices, edge_weights, edge_versors,
           edge_attrs, W1, Wf1, bf1, Wf2, W2, b2, gamma, beta, W3, b3):
    del edge_versors
    s = _lin1(senders[0], W1.T)
    wf1T = Wf1.T.astype(jnp.bfloat16)
    wf2T = Wf2.T.astype(jnp.bfloat16)
    zeros = jnp.zeros((_RPT, _D), jnp.float32)
    packed = edge_indices[0] | (edge_indices[1] << 16)
    pk0 = packed[:_ES].reshape(_NW, _ES // _NW)
    pk1 = packed[_ES:].reshape(_NW, _ES // _NW)
    w0 = _edge_filter(edge_attrs[:_ES], edge_weights[:_ES], wf1T, bf1, wf2T)
    p0 = _SC_SLAB(w0, s, pk0, zeros)
    w1 = _edge_filter(edge_attrs[_ES:], edge_weights[_ES:], wf1T, bf1, wf2T)
    p1 = _SC_SLAB(w1, s, pk1, zeros)
    return _node_mlp(p0, p1, receivers[0], W2.T, b2, gamma, beta, W3.T, b3)


# speedup vs baseline: 1.1449x; 1.1449x over previous
import functools

import jax
import jax.numpy as jnp
from jax import lax
from jax.experimental import pallas as pl
from jax.experimental.pallas import tpu as pltpu
from jax.experimental.pallas import tpu_sc as plsc

_N = 10000
_E = 320000
_D = 128
_R_CUT = 5.0


def _lin1_body(x_ref, wT_ref, o_ref):
    o_ref[...] = jnp.dot(x_ref[...], wT_ref[...],
                         preferred_element_type=jnp.float32)


def _lin1(x, wT):
    return pl.pallas_call(
        _lin1_body,
        out_shape=jax.ShapeDtypeStruct((_N, _D), jnp.float32),
    )(x, wT)


_BE = 1280
_GE = _E // _BE


def _col128(row):
    rid = lax.broadcasted_iota(jnp.int32, (128, 128), 0)
    lid = lax.broadcasted_iota(jnp.int32, (128, 128), 1)
    b = jnp.broadcast_to(row, (128, 128))
    return jnp.sum(jnp.where(rid == lid, b, 0.0), axis=1, keepdims=True)


def _filter_body(attrs_ref, ew_ref, wf1T_ref, bf1_ref, wf2T_ref, w_ref):
    a = attrs_ref[...].astype(jnp.bfloat16)
    h = jnp.dot(a, wf1T_ref[...], preferred_element_type=jnp.float32)
    h = h + bf1_ref[...]
    h = h * jax.nn.sigmoid(h)
    w = jnp.dot(h.astype(jnp.bfloat16), wf2T_ref[...],
                preferred_element_type=jnp.float32)
    ew = ew_ref[0]
    c = 0.5 * (jnp.cos(jnp.pi * ew / _R_CUT) + 1.0)
    c = c * (ew < _R_CUT).astype(jnp.float32)
    ccol = jnp.concatenate(
        [_col128(c[r:r + 1, :]) for r in range(_BE // 128)], axis=0)
    w_ref[...] = w * ccol


def _edge_filter(edge_attrs, edge_weights, wf1T, bf1, wf2T):
    ew2 = edge_weights.reshape(_GE, _BE // 128, 128)
    return pl.pallas_call(
        _filter_body,
        grid=(_GE,),
        in_specs=[
            pl.BlockSpec((_BE, _D), lambda i: (i, 0)),
            pl.BlockSpec((1, _BE // 128, 128), lambda i: (i, 0, 0)),
            pl.BlockSpec((_D, _D), lambda i: (0, 0)),
            pl.BlockSpec((1, _D), lambda i: (0, 0)),
            pl.BlockSpec((_D, _D), lambda i: (0, 0)),
        ],
        out_specs=pl.BlockSpec((_BE, _D), lambda i: (i, 0)),
        out_shape=jax.ShapeDtypeStruct((_E, _D), jnp.float32),
    )(edge_attrs, ew2, wf1T, bf1.reshape(1, _D), wf2T)


_NC, _NS, _L = 2, 16, 16
_NW = _NC * _NS
_CH = 48
_PERW = _E // _NW
_NCH = 208
_TL = _PERW - _NCH * _CH
_NP = 10112
_RPT = _NP // _NS

_sc_mesh = plsc.VectorSubcoreMesh(core_axis_name="c", subcore_axis_name="s")


@functools.partial(
    pl.kernel,
    out_type=jax.ShapeDtypeStruct((_NC, _NP, _D), jnp.float32),
    mesh=_sc_mesh,
    scratch_types=[
        pltpu.VMEM((_PERW,), jnp.int32),
        pltpu.VMEM((_CH,), jnp.int32),
        pltpu.VMEM((_CH,), jnp.int32),
        pltpu.VMEM((_CH,), jnp.int32),
        pltpu.VMEM((_CH,), jnp.int32),
        pltpu.VMEM((_CH,), jnp.int32),
        pltpu.VMEM((_CH,), jnp.int32),
        pltpu.VMEM((_TL,), jnp.int32),
        pltpu.VMEM((_TL,), jnp.int32),
        pltpu.VMEM((_CH, _D), jnp.float32),
        pltpu.VMEM((_CH, _D), jnp.float32),
        pltpu.VMEM((_CH, _D), jnp.float32),
        pltpu.VMEM((_CH, _D), jnp.float32),
        pltpu.VMEM((_CH, _D), jnp.float32),
        pltpu.VMEM((_CH, _D), jnp.float32),
        pltpu.VMEM_SHARED((_NP, _D), jnp.float32),
        pltpu.SemaphoreType.DMA,
        pltpu.SemaphoreType.DMA,
        pltpu.SemaphoreType.DMA,
        pltpu.SemaphoreType.DMA,
        pltpu.SemaphoreType.DMA,
        pltpu.SemaphoreType.DMA,
        pltpu.SemaphoreType.DMA,
        pltpu.SemaphoreType.DMA,
        pltpu.SemaphoreType.DMA,
    ],
)
def _sc_scatter(w_hbm, s_hbm, pk_hbm, z_hbm, out_hbm,
                pk, srcb0, srcb1, srcb2, dstb0, dstb1, dstb2, srct, dstt,
                w0, w1, w2, v0, v1, v2, acc,
                semw0, semw1, semw2, semg0, semg1, semg2,
                sems0, sems1, sems2):
    cid = lax.axis_index("c")
    sid = lax.axis_index("s")
    wid = cid * _NS + sid
    pltpu.sync_copy(z_hbm, acc.at[pl.ds(sid * _RPT, _RPT)])
    pltpu.sync_copy(pk_hbm.at[wid], pk)
    plsc.subcore_barrier()

    mask = jnp.full((_L,), 0xFFFF, jnp.int32)

    def unpack_idx(j, srcb, dstb):
        for k in range(_CH // _L):
            x = pk[pl.ds(j * _CH + k * _L, _L)]
            srcb[pl.ds(k * _L, _L)] = x & mask
            dstb[pl.ds(k * _L, _L)] = lax.shift_right_logical(x, 16)

    def start_loads(j, w_v, v_v, srcb, dstb, semw, semg):
        unpack_idx(j, srcb, dstb)
        base = wid * _PERW + j * _CH
        pltpu.async_copy(w_hbm.at[pl.ds(base, _CH)], w_v, semw)
        pltpu.async_copy(s_hbm.at[srcb], v_v, semg)

    def wait_loads(j, w_v, v_v, srcb, semw, semg):
        base = wid * _PERW + j * _CH
        pltpu.make_async_copy(w_hbm.at[pl.ds(base, _CH)], w_v, semw).wait()
        pltpu.make_async_copy(s_hbm.at[srcb], v_v, semg).wait()

    def multiply(w_v, v_v, nrow):
        def row(r, c2):
            for c in range(_D // _L):
                sl = pl.ds(c * _L, _L)
                v_v[r, sl] = v_v[r, sl] * w_v[r, sl]
            return c2
        lax.fori_loop(0, nrow, row, 0)

    def start_scatter(v_v, dstb, sems):
        pltpu.async_copy(v_v, acc.at[dstb], sems, add=True)

    def wait_scatter(v_v, dstb, sems):
        pltpu.make_async_copy(v_v, acc.at[dstb], sems).wait()

    sets = (
        (srcb0, dstb0, w0, v0, semw0, semg0, sems0),
        (srcb1, dstb1, w1, v1, semw1, semg1, sems1),
        (srcb2, dstb2, w2, v2, semw2, semg2, sems2),
    )

    def process(j, cur, nxt):
        srcb, dstb, w_v, v_v, semw, semg, sems = cur
        srcbq, dstbq, w_q, v_q, semw_q, semg_q, sems_q = nxt

        @pl.when(jnp.logical_and(j >= 1, j + 2 < _NCH))
        def _():
            wait_scatter(v_q, dstbq, sems_q)

        @pl.when(j + 2 < _NCH)
        def _():
            start_loads(j + 2, w_q, v_q, srcbq, dstbq, semw_q, semg_q)

        wait_loads(j, w_v, v_v, srcb, semw, semg)
        multiply(w_v, v_v, _CH)
        start_scatter(v_v, dstb, sems)

    start_loads(0, w0, v0, srcb0, dstb0, semw0, semg0)
    start_loads(1, w1, v1, srcb1, dstb1, semw1, semg1)

    def body(j, carry):
        p = j % 3
        for i in range(3):
            @pl.when(p == i)
            def _(i=i):
                process(j, sets[i], sets[(i + 2) % 3])
        return carry

    lax.fori_loop(0, _NCH, body, 0)
    for c in (_NCH - 3, _NCH - 2, _NCH - 1):
        srcb, dstb, w_v, v_v, semw, semg, sems = sets[c % 3]
        wait_scatter(v_v, dstb, sems)

    xt = pk[pl.ds(_NCH * _CH, _TL)]
    srct[...] = xt & mask
    dstt[...] = lax.shift_right_logical(xt, 16)
    tbase = wid * _PERW + _NCH * _CH
    pltpu.sync_copy(w_hbm.at[pl.ds(tbase, _TL)], w0.at[pl.ds(0, _TL)])
    pltpu.async_copy(s_hbm.at[srct], v0.at[pl.ds(0, _TL)], semg0).wait()
    multiply(w0, v0, _TL)
    pltpu.sync_copy(v0.at[pl.ds(0, _TL)], acc.at[dstt], add=True)

    plsc.subcore_barrier()
    pltpu.sync_copy(acc.at[pl.ds(sid * _RPT, _RPT)],
                    out_hbm.at[cid, pl.ds(sid * _RPT, _RPT)])


_BN = 1000
_GN = _N // _BN


def _node_body(p_ref, recv_ref, w2T_ref, b2_ref, g_ref, be_ref, w3T_ref,
               b3_ref, o_ref):
    upd = p_ref[0] + p_ref[1]
    y = jnp.dot(upd, w2T_ref[...], preferred_element_type=jnp.float32)
    y = y + b2_ref[...]
    mu = jnp.mean(y, axis=-1, keepdims=True)
    yc = y - mu
    var = jnp.mean(yc * yc, axis=-1, keepdims=True)
    y = yc * lax.rsqrt(var + 1e-5) * g_ref[...] + be_ref[...]
    y = y * jax.nn.sigmoid(y)
    o_ref[...] = (jnp.dot(y, w3T_ref[...], preferred_element_type=jnp.float32)
                  + b3_ref[...] + recv_ref[...])


def _node_mlp(partials, recv, w2T, b2, gamma, beta, w3T, b3):
    return pl.pallas_call(
        _node_body,
        grid=(_GN,),
        in_specs=[
            pl.BlockSpec((_NC, _BN, _D), lambda i: (0, i, 0)),
            pl.BlockSpec((_BN, _D), lambda i: (i, 0)),
            pl.BlockSpec((_D, _D), lambda i: (0, 0)),
            pl.BlockSpec((1, _D), lambda i: (0, 0)),
            pl.BlockSpec((1, _D), lambda i: (0, 0)),
            pl.BlockSpec((1, _D), lambda i: (0, 0)),
            pl.BlockSpec((_D, _D), lambda i: (0, 0)),
            pl.BlockSpec((1, _D), lambda i: (0, 0)),
        ],
        out_specs=pl.BlockSpec((_BN, _D), lambda i: (i, 0)),
        out_shape=jax.ShapeDtypeStruct((_N, _D), jnp.float32),
    )(partials, recv, w2T, b2.reshape(1, _D), gamma.reshape(1, _D),
      beta.reshape(1, _D), w3T, b3.reshape(1, _D))


def kernel(senders, receivers, edge_indices, edge_weights, edge_versors,
           edge_attrs, W1, Wf1, bf1, Wf2, W2, b2, gamma, beta, W3, b3):
    del edge_versors
    s = _lin1(senders[0], W1.T)
    w = _edge_filter(edge_attrs, edge_weights,
                     Wf1.T.astype(jnp.bfloat16), bf1,
                     Wf2.T.astype(jnp.bfloat16))
    zeros = jnp.zeros((_RPT, _D), jnp.float32)
    packed = (edge_indices[0] | (edge_indices[1] << 16)).reshape(_NW, _PERW)
    partials = _sc_scatter(w, s, packed, zeros)
    return _node_mlp(partials, receivers[0], W2.T, b2, gamma, beta,
                     W3.T, b3)


# speedup vs baseline: 1.3409x; 1.1711x over previous
import functools

import jax
import jax.numpy as jnp
from jax import lax
from jax.experimental import pallas as pl
from jax.experimental.pallas import tpu as pltpu
from jax.experimental.pallas import tpu_sc as plsc

_N = 10000
_E = 320000
_D = 128
_R_CUT = 5.0


def _lin1_body(x_ref, wT_ref, o_ref):
    o_ref[...] = jnp.dot(x_ref[...], wT_ref[...],
                         preferred_element_type=jnp.float32)


def _lin1(x, wT):
    return pl.pallas_call(
        _lin1_body,
        out_shape=jax.ShapeDtypeStruct((_N, _D), jnp.float32),
    )(x, wT)


_BE = 2560
_GE = _E // _BE


def _col128(row):
    rid = lax.broadcasted_iota(jnp.int32, (128, 128), 0)
    lid = lax.broadcasted_iota(jnp.int32, (128, 128), 1)
    b = jnp.broadcast_to(row, (128, 128))
    return jnp.sum(jnp.where(rid == lid, b, 0.0), axis=1, keepdims=True)


def _filter_body(attrs_ref, ew_ref, wf1T_ref, bf1_ref, wf2T_ref, w_ref):
    a = attrs_ref[...].astype(jnp.bfloat16)
    h = jnp.dot(a, wf1T_ref[...], preferred_element_type=jnp.float32)
    h = h + bf1_ref[...]
    h = h * jax.nn.sigmoid(h)
    w = jnp.dot(h.astype(jnp.bfloat16), wf2T_ref[...],
                preferred_element_type=jnp.float32)
    ew = ew_ref[0]
    c = 0.5 * (jnp.cos(jnp.pi * ew / _R_CUT) + 1.0)
    c = c * (ew < _R_CUT).astype(jnp.float32)
    ccol = jnp.concatenate(
        [_col128(c[r:r + 1, :]) for r in range(_BE // 128)], axis=0)
    w_ref[...] = w * ccol


def _edge_filter(edge_attrs, edge_weights, wf1T, bf1, wf2T):
    ew2 = edge_weights.reshape(_GE, _BE // 128, 128)
    return pl.pallas_call(
        _filter_body,
        grid=(_GE,),
        in_specs=[
            pl.BlockSpec((_BE, _D), lambda i: (i, 0)),
            pl.BlockSpec((1, _BE // 128, 128), lambda i: (i, 0, 0)),
            pl.BlockSpec((_D, _D), lambda i: (0, 0)),
            pl.BlockSpec((1, _D), lambda i: (0, 0)),
            pl.BlockSpec((_D, _D), lambda i: (0, 0)),
        ],
        out_specs=pl.BlockSpec((_BE, _D), lambda i: (i, 0)),
        out_shape=jax.ShapeDtypeStruct((_E, _D), jnp.float32),
    )(edge_attrs, ew2, wf1T, bf1.reshape(1, _D), wf2T)


_NC, _NS, _L = 2, 16, 16
_NW = _NC * _NS
_CH = 48
_PERW = _E // _NW
_NCH = 208
_TL = _PERW - _NCH * _CH
_NP = 10112
_RPT = _NP // _NS

_sc_mesh = plsc.VectorSubcoreMesh(core_axis_name="c", subcore_axis_name="s")


@functools.partial(
    pl.kernel,
    out_type=jax.ShapeDtypeStruct((_NC, _NP, _D), jnp.float32),
    mesh=_sc_mesh,
    scratch_types=[
        pltpu.VMEM((_PERW,), jnp.int32),
        pltpu.VMEM((_CH,), jnp.int32),
        pltpu.VMEM((_CH,), jnp.int32),
        pltpu.VMEM((_CH,), jnp.int32),
        pltpu.VMEM((_CH,), jnp.int32),
        pltpu.VMEM((_CH,), jnp.int32),
        pltpu.VMEM((_CH,), jnp.int32),
        pltpu.VMEM((_TL,), jnp.int32),
        pltpu.VMEM((_TL,), jnp.int32),
        pltpu.VMEM((_CH, _D), jnp.float32),
        pltpu.VMEM((_CH, _D), jnp.float32),
        pltpu.VMEM((_CH, _D), jnp.float32),
        pltpu.VMEM((_CH, _D), jnp.float32),
        pltpu.VMEM((_CH, _D), jnp.float32),
        pltpu.VMEM((_CH, _D), jnp.float32),
        pltpu.VMEM_SHARED((_NP, _D), jnp.float32),
        pltpu.SemaphoreType.DMA,
        pltpu.SemaphoreType.DMA,
        pltpu.SemaphoreType.DMA,
        pltpu.SemaphoreType.DMA,
        pltpu.SemaphoreType.DMA,
        pltpu.SemaphoreType.DMA,
        pltpu.SemaphoreType.DMA,
        pltpu.SemaphoreType.DMA,
        pltpu.SemaphoreType.DMA,
    ],
)
def _sc_scatter(w_hbm, s_hbm, pk_hbm, z_hbm, out_hbm,
                pk, srcb0, srcb1, srcb2, dstb0, dstb1, dstb2, srct, dstt,
                w0, w1, w2, v0, v1, v2, acc,
                semw0, semw1, semw2, semg0, semg1, semg2,
                sems0, sems1, sems2):
    cid = lax.axis_index("c")
    sid = lax.axis_index("s")
    wid = cid * _NS + sid
    pltpu.sync_copy(z_hbm, acc.at[pl.ds(sid * _RPT, _RPT)])
    pltpu.sync_copy(pk_hbm.at[wid], pk)
    plsc.subcore_barrier()

    mask = jnp.full((_L,), 0xFFFF, jnp.int32)

    def unpack_idx(j, srcb, dstb):
        for k in range(_CH // _L):
            x = pk[pl.ds(j * _CH + k * _L, _L)]
            srcb[pl.ds(k * _L, _L)] = x & mask
            dstb[pl.ds(k * _L, _L)] = lax.shift_right_logical(x, 16)

    def start_loads(j, w_v, v_v, srcb, dstb, semw, semg):
        unpack_idx(j, srcb, dstb)
        base = wid * _PERW + j * _CH
        pltpu.async_copy(w_hbm.at[pl.ds(base, _CH)], w_v, semw)
        pltpu.async_copy(s_hbm.at[srcb], v_v, semg)

    def wait_loads(j, w_v, v_v, srcb, semw, semg):
        base = wid * _PERW + j * _CH
        pltpu.make_async_copy(w_hbm.at[pl.ds(base, _CH)], w_v, semw).wait()
        pltpu.make_async_copy(s_hbm.at[srcb], v_v, semg).wait()

    def multiply(w_v, v_v, nrow):
        def row(r, c2):
            for c in range(_D // _L):
                sl = pl.ds(c * _L, _L)
                v_v[r, sl] = v_v[r, sl] * w_v[r, sl]
            return c2
        lax.fori_loop(0, nrow, row, 0)

    def start_scatter(v_v, dstb, sems):
        pltpu.async_copy(v_v, acc.at[dstb], sems, add=True)

    def wait_scatter(v_v, dstb, sems):
        pltpu.make_async_copy(v_v, acc.at[dstb], sems).wait()

    sets = (
        (srcb0, dstb0, w0, v0, semw0, semg0, sems0),
        (srcb1, dstb1, w1, v1, semw1, semg1, sems1),
        (srcb2, dstb2, w2, v2, semw2, semg2, sems2),
    )

    def process(j, cur, nxt):
        srcb, dstb, w_v, v_v, semw, semg, sems = cur
        srcbq, dstbq, w_q, v_q, semw_q, semg_q, sems_q = nxt

        @pl.when(jnp.logical_and(j >= 1, j + 2 < _NCH))
        def _():
            wait_scatter(v_q, dstbq, sems_q)

        @pl.when(j + 2 < _NCH)
        def _():
            start_loads(j + 2, w_q, v_q, srcbq, dstbq, semw_q, semg_q)

        wait_loads(j, w_v, v_v, srcb, semw, semg)
        multiply(w_v, v_v, _CH)
        start_scatter(v_v, dstb, sems)

    start_loads(0, w0, v0, srcb0, dstb0, semw0, semg0)
    start_loads(1, w1, v1, srcb1, dstb1, semw1, semg1)

    def body(j, carry):
        p = j % 3
        for i in range(3):
            @pl.when(p == i)
            def _(i=i):
                process(j, sets[i], sets[(i + 2) % 3])
        return carry

    lax.fori_loop(0, _NCH, body, 0)
    for c in (_NCH - 3, _NCH - 2, _NCH - 1):
        srcb, dstb, w_v, v_v, semw, semg, sems = sets[c % 3]
        wait_scatter(v_v, dstb, sems)

    xt = pk[pl.ds(_NCH * _CH, _TL)]
    srct[...] = xt & mask
    dstt[...] = lax.shift_right_logical(xt, 16)
    tbase = wid * _PERW + _NCH * _CH
    pltpu.sync_copy(w_hbm.at[pl.ds(tbase, _TL)], w0.at[pl.ds(0, _TL)])
    pltpu.async_copy(s_hbm.at[srct], v0.at[pl.ds(0, _TL)], semg0).wait()
    multiply(w0, v0, _TL)
    pltpu.sync_copy(v0.at[pl.ds(0, _TL)], acc.at[dstt], add=True)

    plsc.subcore_barrier()
    pltpu.sync_copy(acc.at[pl.ds(sid * _RPT, _RPT)],
                    out_hbm.at[cid, pl.ds(sid * _RPT, _RPT)])


_BN = 1000
_GN = _N // _BN


def _node_body(p_ref, recv_ref, w2T_ref, b2_ref, g_ref, be_ref, w3T_ref,
               b3_ref, o_ref):
    upd = p_ref[0] + p_ref[1]
    y = jnp.dot(upd, w2T_ref[...], preferred_element_type=jnp.float32)
    y = y + b2_ref[...]
    mu = jnp.mean(y, axis=-1, keepdims=True)
    yc = y - mu
    var = jnp.mean(yc * yc, axis=-1, keepdims=True)
    y = yc * lax.rsqrt(var + 1e-5) * g_ref[...] + be_ref[...]
    y = y * jax.nn.sigmoid(y)
    o_ref[...] = (jnp.dot(y, w3T_ref[...], preferred_element_type=jnp.float32)
                  + b3_ref[...] + recv_ref[...])


def _node_mlp(partials, recv, w2T, b2, gamma, beta, w3T, b3):
    return pl.pallas_call(
        _node_body,
        grid=(_GN,),
        in_specs=[
            pl.BlockSpec((_NC, _BN, _D), lambda i: (0, i, 0)),
            pl.BlockSpec((_BN, _D), lambda i: (i, 0)),
            pl.BlockSpec((_D, _D), lambda i: (0, 0)),
            pl.BlockSpec((1, _D), lambda i: (0, 0)),
            pl.BlockSpec((1, _D), lambda i: (0, 0)),
            pl.BlockSpec((1, _D), lambda i: (0, 0)),
            pl.BlockSpec((_D, _D), lambda i: (0, 0)),
            pl.BlockSpec((1, _D), lambda i: (0, 0)),
        ],
        out_specs=pl.BlockSpec((_BN, _D), lambda i: (i, 0)),
        out_shape=jax.ShapeDtypeStruct((_N, _D), jnp.float32),
    )(partials, recv, w2T, b2.reshape(1, _D), gamma.reshape(1, _D),
      beta.reshape(1, _D), w3T, b3.reshape(1, _D))


def kernel(senders, receivers, edge_indices, edge_weights, edge_versors,
           edge_attrs, W1, Wf1, bf1, Wf2, W2, b2, gamma, beta, W3, b3):
    del edge_versors
    s = _lin1(senders[0], W1.T)
    w = _edge_filter(edge_attrs, edge_weights,
                     Wf1.T.astype(jnp.bfloat16), bf1,
                     Wf2.T.astype(jnp.bfloat16))
    zeros = jnp.zeros((_RPT, _D), jnp.float32)
    packed = (edge_indices[0] | (edge_indices[1] << 16)).reshape(_NW, _PERW)
    partials = _sc_scatter(w, s, packed, zeros)
    return _node_mlp(partials, receivers[0], W2.T, b2, gamma, beta,
                     W3.T, b3)


# speedup vs baseline: 1.3939x; 1.0396x over previous
import functools

import jax
import jax.numpy as jnp
from jax import lax
from jax.experimental import pallas as pl
from jax.experimental.pallas import tpu as pltpu
from jax.experimental.pallas import tpu_sc as plsc

_N = 10000
_E = 320000
_D = 128
_R_CUT = 5.0


def _lin1_body(x_ref, wT_ref, o_ref):
    o_ref[...] = jnp.dot(x_ref[...], wT_ref[...],
                         preferred_element_type=jnp.float32)


def _lin1(x, wT):
    return pl.pallas_call(
        _lin1_body,
        out_shape=jax.ShapeDtypeStruct((_N, _D), jnp.float32),
    )(x, wT)


_BE = 3200
_GE = _E // _BE


def _col128(row):
    rid = lax.broadcasted_iota(jnp.int32, (128, 128), 0)
    lid = lax.broadcasted_iota(jnp.int32, (128, 128), 1)
    b = jnp.broadcast_to(row, (128, 128))
    return jnp.sum(jnp.where(rid == lid, b, 0.0), axis=1, keepdims=True)


def _filter_body(attrs_ref, ew_ref, wf1T_ref, bf1_ref, wf2T_ref, w_ref):
    a = attrs_ref[...].astype(jnp.bfloat16)
    h = jnp.dot(a, wf1T_ref[...], preferred_element_type=jnp.float32)
    h = h + bf1_ref[...]
    h = h * jax.nn.sigmoid(h)
    w = jnp.dot(h.astype(jnp.bfloat16), wf2T_ref[...],
                preferred_element_type=jnp.float32)
    ew = ew_ref[0]
    c = 0.5 * (jnp.cos(jnp.pi * ew / _R_CUT) + 1.0)
    c = c * (ew < _R_CUT).astype(jnp.float32)
    ccol = jnp.concatenate(
        [_col128(c[r:r + 1, :]) for r in range(_BE // 128)], axis=0)
    w_ref[...] = w * ccol


def _edge_filter(edge_attrs, edge_weights, wf1T, bf1, wf2T):
    ew2 = edge_weights.reshape(_GE, _BE // 128, 128)
    return pl.pallas_call(
        _filter_body,
        grid=(_GE,),
        in_specs=[
            pl.BlockSpec((_BE, _D), lambda i: (i, 0)),
            pl.BlockSpec((1, _BE // 128, 128), lambda i: (i, 0, 0)),
            pl.BlockSpec((_D, _D), lambda i: (0, 0)),
            pl.BlockSpec((1, _D), lambda i: (0, 0)),
            pl.BlockSpec((_D, _D), lambda i: (0, 0)),
        ],
        out_specs=pl.BlockSpec((_BE, _D), lambda i: (i, 0)),
        out_shape=jax.ShapeDtypeStruct((_E, _D), jnp.float32),
    )(edge_attrs, ew2, wf1T, bf1.reshape(1, _D), wf2T)


_NC, _NS, _L = 2, 16, 16
_NW = _NC * _NS
_CH = 48
_PERW = _E // _NW
_NCH = 208
_TL = _PERW - _NCH * _CH
_NP = 10112
_RPT = _NP // _NS

_sc_mesh = plsc.VectorSubcoreMesh(core_axis_name="c", subcore_axis_name="s")


@functools.partial(
    pl.kernel,
    out_type=jax.ShapeDtypeStruct((_NC, _NP, _D), jnp.float32),
    mesh=_sc_mesh,
    scratch_types=[
        pltpu.VMEM((_PERW,), jnp.int32),
        pltpu.VMEM((_CH,), jnp.int32),
        pltpu.VMEM((_CH,), jnp.int32),
        pltpu.VMEM((_CH,), jnp.int32),
        pltpu.VMEM((_CH,), jnp.int32),
        pltpu.VMEM((_CH,), jnp.int32),
        pltpu.VMEM((_CH,), jnp.int32),
        pltpu.VMEM((_TL,), jnp.int32),
        pltpu.VMEM((_TL,), jnp.int32),
        pltpu.VMEM((_CH, _D), jnp.float32),
        pltpu.VMEM((_CH, _D), jnp.float32),
        pltpu.VMEM((_CH, _D), jnp.float32),
        pltpu.VMEM((_CH, _D), jnp.float32),
        pltpu.VMEM((_CH, _D), jnp.float32),
        pltpu.VMEM((_CH, _D), jnp.float32),
        pltpu.VMEM_SHARED((_NP, _D), jnp.float32),
        pltpu.SemaphoreType.DMA,
        pltpu.SemaphoreType.DMA,
        pltpu.SemaphoreType.DMA,
        pltpu.SemaphoreType.DMA,
        pltpu.SemaphoreType.DMA,
        pltpu.SemaphoreType.DMA,
        pltpu.SemaphoreType.DMA,
        pltpu.SemaphoreType.DMA,
        pltpu.SemaphoreType.DMA,
    ],
)
def _sc_scatter(w_hbm, s_hbm, pk_hbm, z_hbm, out_hbm,
                pk, srcb0, srcb1, srcb2, dstb0, dstb1, dstb2, srct, dstt,
                w0, w1, w2, v0, v1, v2, acc,
                semw0, semw1, semw2, semg0, semg1, semg2,
                sems0, sems1, sems2):
    cid = lax.axis_index("c")
    sid = lax.axis_index("s")
    wid = cid * _NS + sid
    pltpu.sync_copy(z_hbm, acc.at[pl.ds(sid * _RPT, _RPT)])
    pltpu.sync_copy(pk_hbm.at[wid], pk)
    plsc.subcore_barrier()

    mask = jnp.full((_L,), 0xFFFF, jnp.int32)

    def unpack_idx(j, srcb, dstb):
        for k in range(_CH // _L):
            x = pk[pl.ds(j * _CH + k * _L, _L)]
            srcb[pl.ds(k * _L, _L)] = x & mask
            dstb[pl.ds(k * _L, _L)] = lax.shift_right_logical(x, 16)

    def start_loads(j, w_v, v_v, srcb, dstb, semw, semg):
        unpack_idx(j, srcb, dstb)
        base = wid * _PERW + j * _CH
        pltpu.async_copy(w_hbm.at[pl.ds(base, _CH)], w_v, semw)
        pltpu.async_copy(s_hbm.at[srcb], v_v, semg)

    def wait_loads(j, w_v, v_v, srcb, semw, semg):
        base = wid * _PERW + j * _CH
        pltpu.make_async_copy(w_hbm.at[pl.ds(base, _CH)], w_v, semw).wait()
        pltpu.make_async_copy(s_hbm.at[srcb], v_v, semg).wait()

    def multiply(w_v, v_v, nrow):
        def row(r, c2):
            for c in range(_D // _L):
                sl = pl.ds(c * _L, _L)
                v_v[r, sl] = v_v[r, sl] * w_v[r, sl]
            return c2
        lax.fori_loop(0, nrow, row, 0)

    def start_scatter(v_v, dstb, sems):
        pltpu.async_copy(v_v, acc.at[dstb], sems, add=True)

    def wait_scatter(v_v, dstb, sems):
        pltpu.make_async_copy(v_v, acc.at[dstb], sems).wait()

    sets = (
        (srcb0, dstb0, w0, v0, semw0, semg0, sems0),
        (srcb1, dstb1, w1, v1, semw1, semg1, sems1),
        (srcb2, dstb2, w2, v2, semw2, semg2, sems2),
    )

    def process(j, cur, nxt):
        srcb, dstb, w_v, v_v, semw, semg, sems = cur
        srcbq, dstbq, w_q, v_q, semw_q, semg_q, sems_q = nxt

        @pl.when(jnp.logical_and(j >= 1, j + 2 < _NCH))
        def _():
            wait_scatter(v_q, dstbq, sems_q)

        @pl.when(j + 2 < _NCH)
        def _():
            start_loads(j + 2, w_q, v_q, srcbq, dstbq, semw_q, semg_q)

        wait_loads(j, w_v, v_v, srcb, semw, semg)
        multiply(w_v, v_v, _CH)
        start_scatter(v_v, dstb, sems)

    start_loads(0, w0, v0, srcb0, dstb0, semw0, semg0)
    start_loads(1, w1, v1, srcb1, dstb1, semw1, semg1)

    def body(j, carry):
        p = j % 3
        for i in range(3):
            @pl.when(p == i)
            def _(i=i):
                process(j, sets[i], sets[(i + 2) % 3])
        return carry

    lax.fori_loop(0, _NCH, body, 0)
    for c in (_NCH - 3, _NCH - 2, _NCH - 1):
        srcb, dstb, w_v, v_v, semw, semg, sems = sets[c % 3]
        wait_scatter(v_v, dstb, sems)

    xt = pk[pl.ds(_NCH * _CH, _TL)]
    srct[...] = xt & mask
    dstt[...] = lax.shift_right_logical(xt, 16)
    tbase = wid * _PERW + _NCH * _CH
    pltpu.sync_copy(w_hbm.at[pl.ds(tbase, _TL)], w0.at[pl.ds(0, _TL)])
    pltpu.async_copy(s_hbm.at[srct], v0.at[pl.ds(0, _TL)], semg0).wait()
    multiply(w0, v0, _TL)
    pltpu.sync_copy(v0.at[pl.ds(0, _TL)], acc.at[dstt], add=True)

    plsc.subcore_barrier()
    pltpu.sync_copy(acc.at[pl.ds(sid * _RPT, _RPT)],
                    out_hbm.at[cid, pl.ds(sid * _RPT, _RPT)])


_BN = 1000
_GN = _N // _BN


def _node_body(p_ref, recv_ref, w2T_ref, b2_ref, g_ref, be_ref, w3T_ref,
               b3_ref, o_ref):
    upd = p_ref[0] + p_ref[1]
    y = jnp.dot(upd, w2T_ref[...], preferred_element_type=jnp.float32)
    y = y + b2_ref[...]
    mu = jnp.mean(y, axis=-1, keepdims=True)
    yc = y - mu
    var = jnp.mean(yc * yc, axis=-1, keepdims=True)
    y = yc * lax.rsqrt(var + 1e-5) * g_ref[...] + be_ref[...]
    y = y * jax.nn.sigmoid(y)
    o_ref[...] = (jnp.dot(y, w3T_ref[...], preferred_element_type=jnp.float32)
                  + b3_ref[...] + recv_ref[...])


def _node_mlp(partials, recv, w2T, b2, gamma, beta, w3T, b3):
    return pl.pallas_call(
        _node_body,
        grid=(_GN,),
        in_specs=[
            pl.BlockSpec((_NC, _BN, _D), lambda i: (0, i, 0)),
            pl.BlockSpec((_BN, _D), lambda i: (i, 0)),
            pl.BlockSpec((_D, _D), lambda i: (0, 0)),
            pl.BlockSpec((1, _D), lambda i: (0, 0)),
            pl.BlockSpec((1, _D), lambda i: (0, 0)),
            pl.BlockSpec((1, _D), lambda i: (0, 0)),
            pl.BlockSpec((_D, _D), lambda i: (0, 0)),
            pl.BlockSpec((1, _D), lambda i: (0, 0)),
        ],
        out_specs=pl.BlockSpec((_BN, _D), lambda i: (i, 0)),
        out_shape=jax.ShapeDtypeStruct((_N, _D), jnp.float32),
    )(partials, recv, w2T, b2.reshape(1, _D), gamma.reshape(1, _D),
      beta.reshape(1, _D), w3T, b3.reshape(1, _D))


def kernel(senders, receivers, edge_indices, edge_weights, edge_versors,
           edge_attrs, W1, Wf1, bf1, Wf2, W2, b2, gamma, beta, W3, b3):
    del edge_versors
    s = _lin1(senders[0], W1.T)
    w = _edge_filter(edge_attrs, edge_weights,
                     Wf1.T.astype(jnp.bfloat16), bf1,
                     Wf2.T.astype(jnp.bfloat16))
    zeros = jnp.zeros((_RPT, _D), jnp.float32)
    packed = (edge_indices[0] | (edge_indices[1] << 16)).reshape(_NW, _PERW)
    partials = _sc_scatter(w, s, packed, zeros)
    return _node_mlp(partials, receivers[0], W2.T, b2, gamma, beta,
                     W3.T, b3)


# speedup vs baseline: 1.5075x; 1.0815x over previous
import functools

import jax
import jax.numpy as jnp
from jax import lax
from jax.experimental import pallas as pl
from jax.experimental.pallas import tpu as pltpu
from jax.experimental.pallas import tpu_sc as plsc

_N = 10000
_E = 320000
_D = 128
_R_CUT = 5.0


def _lin1_body(x_ref, wT_ref, o_ref):
    o_ref[...] = jnp.dot(x_ref[...], wT_ref[...],
                         preferred_element_type=jnp.float32)


def _lin1(x, wT):
    return pl.pallas_call(
        _lin1_body,
        out_shape=jax.ShapeDtypeStruct((_N, _D), jnp.float32),
    )(x, wT)


_BE = 6400
_GE = _E // _BE


def _col128(row):
    rid = lax.broadcasted_iota(jnp.int32, (128, 128), 0)
    lid = lax.broadcasted_iota(jnp.int32, (128, 128), 1)
    b = jnp.broadcast_to(row, (128, 128))
    return jnp.sum(jnp.where(rid == lid, b, 0.0), axis=1, keepdims=True)


def _filter_body(attrs_ref, ew_ref, wf1T_ref, bf1_ref, wf2T_ref, w_ref):
    a = attrs_ref[...].astype(jnp.bfloat16)
    h = jnp.dot(a, wf1T_ref[...], preferred_element_type=jnp.float32)
    h = h + bf1_ref[...]
    h = h * jax.nn.sigmoid(h)
    w = jnp.dot(h.astype(jnp.bfloat16), wf2T_ref[...],
                preferred_element_type=jnp.float32)
    ew = ew_ref[0]
    c = 0.5 * (jnp.cos(jnp.pi * ew / _R_CUT) + 1.0)
    c = c * (ew < _R_CUT).astype(jnp.float32)
    ccol = jnp.concatenate(
        [_col128(c[r:r + 1, :]) for r in range(_BE // 128)], axis=0)
    w_ref[...] = w * ccol


def _edge_filter(edge_attrs, edge_weights, wf1T, bf1, wf2T):
    ew2 = edge_weights.reshape(_GE, _BE // 128, 128)
    return pl.pallas_call(
        _filter_body,
        grid=(_GE,),
        in_specs=[
            pl.BlockSpec((_BE, _D), lambda i: (i, 0)),
            pl.BlockSpec((1, _BE // 128, 128), lambda i: (i, 0, 0)),
            pl.BlockSpec((_D, _D), lambda i: (0, 0)),
            pl.BlockSpec((1, _D), lambda i: (0, 0)),
            pl.BlockSpec((_D, _D), lambda i: (0, 0)),
        ],
        out_specs=pl.BlockSpec((_BE, _D), lambda i: (i, 0)),
        out_shape=jax.ShapeDtypeStruct((_E, _D), jnp.float32),
    )(edge_attrs, ew2, wf1T, bf1.reshape(1, _D), wf2T)


_NC, _NS, _L = 2, 16, 16
_NW = _NC * _NS
_CH = 48
_PERW = _E // _NW
_NCH = 208
_TL = _PERW - _NCH * _CH
_NP = 10112
_RPT = _NP // _NS

_sc_mesh = plsc.VectorSubcoreMesh(core_axis_name="c", subcore_axis_name="s")


@functools.partial(
    pl.kernel,
    out_type=jax.ShapeDtypeStruct((_NC, _NP, _D), jnp.float32),
    mesh=_sc_mesh,
    scratch_types=[
        pltpu.VMEM((_PERW,), jnp.int32),
        pltpu.VMEM((_CH,), jnp.int32),
        pltpu.VMEM((_CH,), jnp.int32),
        pltpu.VMEM((_CH,), jnp.int32),
        pltpu.VMEM((_CH,), jnp.int32),
        pltpu.VMEM((_CH,), jnp.int32),
        pltpu.VMEM((_CH,), jnp.int32),
        pltpu.VMEM((_TL,), jnp.int32),
        pltpu.VMEM((_TL,), jnp.int32),
        pltpu.VMEM((_CH, _D), jnp.float32),
        pltpu.VMEM((_CH, _D), jnp.float32),
        pltpu.VMEM((_CH, _D), jnp.float32),
        pltpu.VMEM((_CH, _D), jnp.float32),
        pltpu.VMEM((_CH, _D), jnp.float32),
        pltpu.VMEM((_CH, _D), jnp.float32),
        pltpu.VMEM_SHARED((_NP, _D), jnp.float32),
        pltpu.SemaphoreType.DMA,
        pltpu.SemaphoreType.DMA,
        pltpu.SemaphoreType.DMA,
        pltpu.SemaphoreType.DMA,
        pltpu.SemaphoreType.DMA,
        pltpu.SemaphoreType.DMA,
        pltpu.SemaphoreType.DMA,
        pltpu.SemaphoreType.DMA,
        pltpu.SemaphoreType.DMA,
    ],
)
def _sc_scatter(w_hbm, s_hbm, pk_hbm, z_hbm, out_hbm,
                pk, srcb0, srcb1, srcb2, dstb0, dstb1, dstb2, srct, dstt,
                w0, w1, w2, v0, v1, v2, acc,
                semw0, semw1, semw2, semg0, semg1, semg2,
                sems0, sems1, sems2):
    cid = lax.axis_index("c")
    sid = lax.axis_index("s")
    wid = cid * _NS + sid
    pltpu.sync_copy(z_hbm, acc.at[pl.ds(sid * _RPT, _RPT)])
    pltpu.sync_copy(pk_hbm.at[wid], pk)
    plsc.subcore_barrier()

    mask = jnp.full((_L,), 0xFFFF, jnp.int32)

    def unpack_idx(j, srcb, dstb):
        for k in range(_CH // _L):
            x = pk[pl.ds(j * _CH + k * _L, _L)]
            srcb[pl.ds(k * _L, _L)] = x & mask
            dstb[pl.ds(k * _L, _L)] = lax.shift_right_logical(x, 16)

    def start_loads(j, w_v, v_v, srcb, dstb, semw, semg):
        unpack_idx(j, srcb, dstb)
        base = wid * _PERW + j * _CH
        pltpu.async_copy(w_hbm.at[pl.ds(base, _CH)], w_v, semw)
        pltpu.async_copy(s_hbm.at[srcb], v_v, semg)

    def wait_loads(j, w_v, v_v, srcb, semw, semg):
        base = wid * _PERW + j * _CH
        pltpu.make_async_copy(w_hbm.at[pl.ds(base, _CH)], w_v, semw).wait()
        pltpu.make_async_copy(s_hbm.at[srcb], v_v, semg).wait()

    def multiply(w_v, v_v, nrow):
        def row(r, c2):
            for c in range(_D // _L):
                sl = pl.ds(c * _L, _L)
                v_v[r, sl] = v_v[r, sl] * w_v[r, sl]
            return c2
        lax.fori_loop(0, nrow, row, 0)

    def start_scatter(v_v, dstb, sems):
        pltpu.async_copy(v_v, acc.at[dstb], sems, add=True)

    def wait_scatter(v_v, dstb, sems):
        pltpu.make_async_copy(v_v, acc.at[dstb], sems).wait()

    sets = (
        (srcb0, dstb0, w0, v0, semw0, semg0, sems0),
        (srcb1, dstb1, w1, v1, semw1, semg1, sems1),
        (srcb2, dstb2, w2, v2, semw2, semg2, sems2),
    )

    def process(j, cur, nxt):
        srcb, dstb, w_v, v_v, semw, semg, sems = cur
        srcbq, dstbq, w_q, v_q, semw_q, semg_q, sems_q = nxt

        @pl.when(jnp.logical_and(j >= 1, j + 2 < _NCH))
        def _():
            wait_scatter(v_q, dstbq, sems_q)

        @pl.when(j + 2 < _NCH)
        def _():
            start_loads(j + 2, w_q, v_q, srcbq, dstbq, semw_q, semg_q)

        wait_loads(j, w_v, v_v, srcb, semw, semg)
        multiply(w_v, v_v, _CH)
        start_scatter(v_v, dstb, sems)

    start_loads(0, w0, v0, srcb0, dstb0, semw0, semg0)
    start_loads(1, w1, v1, srcb1, dstb1, semw1, semg1)

    def body(j, carry):
        p = j % 3
        for i in range(3):
            @pl.when(p == i)
            def _(i=i):
                process(j, sets[i], sets[(i + 2) % 3])
        return carry

    lax.fori_loop(0, _NCH, body, 0)
    for c in (_NCH - 3, _NCH - 2, _NCH - 1):
        srcb, dstb, w_v, v_v, semw, semg, sems = sets[c % 3]
        wait_scatter(v_v, dstb, sems)

    xt = pk[pl.ds(_NCH * _CH, _TL)]
    srct[...] = xt & mask
    dstt[...] = lax.shift_right_logical(xt, 16)
    tbase = wid * _PERW + _NCH * _CH
    pltpu.sync_copy(w_hbm.at[pl.ds(tbase, _TL)], w0.at[pl.ds(0, _TL)])
    pltpu.async_copy(s_hbm.at[srct], v0.at[pl.ds(0, _TL)], semg0).wait()
    multiply(w0, v0, _TL)
    pltpu.sync_copy(v0.at[pl.ds(0, _TL)], acc.at[dstt], add=True)

    plsc.subcore_barrier()
    pltpu.sync_copy(acc.at[pl.ds(sid * _RPT, _RPT)],
                    out_hbm.at[cid, pl.ds(sid * _RPT, _RPT)])


_BN = 1000
_GN = _N // _BN


def _node_body(p_ref, recv_ref, w2T_ref, b2_ref, g_ref, be_ref, w3T_ref,
               b3_ref, o_ref):
    upd = p_ref[0] + p_ref[1]
    y = jnp.dot(upd, w2T_ref[...], preferred_element_type=jnp.float32)
    y = y + b2_ref[...]
    mu = jnp.mean(y, axis=-1, keepdims=True)
    yc = y - mu
    var = jnp.mean(yc * yc, axis=-1, keepdims=True)
    y = yc * lax.rsqrt(var + 1e-5) * g_ref[...] + be_ref[...]
    y = y * jax.nn.sigmoid(y)
    o_ref[...] = (jnp.dot(y, w3T_ref[...], preferred_element_type=jnp.float32)
                  + b3_ref[...] + recv_ref[...])


def _node_mlp(partials, recv, w2T, b2, gamma, beta, w3T, b3):
    return pl.pallas_call(
        _node_body,
        grid=(_GN,),
        in_specs=[
            pl.BlockSpec((_NC, _BN, _D), lambda i: (0, i, 0)),
            pl.BlockSpec((_BN, _D), lambda i: (i, 0)),
            pl.BlockSpec((_D, _D), lambda i: (0, 0)),
            pl.BlockSpec((1, _D), lambda i: (0, 0)),
            pl.BlockSpec((1, _D), lambda i: (0, 0)),
            pl.BlockSpec((1, _D), lambda i: (0, 0)),
            pl.BlockSpec((_D, _D), lambda i: (0, 0)),
            pl.BlockSpec((1, _D), lambda i: (0, 0)),
        ],
        out_specs=pl.BlockSpec((_BN, _D), lambda i: (i, 0)),
        out_shape=jax.ShapeDtypeStruct((_N, _D), jnp.float32),
    )(partials, recv, w2T, b2.reshape(1, _D), gamma.reshape(1, _D),
      beta.reshape(1, _D), w3T, b3.reshape(1, _D))


def kernel(senders, receivers, edge_indices, edge_weights, edge_versors,
           edge_attrs, W1, Wf1, bf1, Wf2, W2, b2, gamma, beta, W3, b3):
    del edge_versors
    s = _lin1(senders[0], W1.T)
    w = _edge_filter(edge_attrs, edge_weights,
                     Wf1.T.astype(jnp.bfloat16), bf1,
                     Wf2.T.astype(jnp.bfloat16))
    zeros = jnp.zeros((_RPT, _D), jnp.float32)
    packed = (edge_indices[0] | (edge_indices[1] << 16)).reshape(_NW, _PERW)
    partials = _sc_scatter(w, s, packed, zeros)
    return _node_mlp(partials, receivers[0], W2.T, b2, gamma, beta,
                     W3.T, b3)


# speedup vs baseline: 1.5753x; 1.0450x over previous
import functools

import jax
import jax.numpy as jnp
from jax import lax
from jax.experimental import pallas as pl
from jax.experimental.pallas import tpu as pltpu
from jax.experimental.pallas import tpu_sc as plsc

_N = 10000
_E = 320000
_D = 128
_R_CUT = 5.0


def _lin1_body(x_ref, wT_ref, o_ref):
    o_ref[...] = jnp.dot(x_ref[...], wT_ref[...],
                         preferred_element_type=jnp.float32)


def _lin1(x, wT):
    return pl.pallas_call(
        _lin1_body,
        out_shape=jax.ShapeDtypeStruct((_N, _D), jnp.float32),
    )(x, wT)


_BE = 12800
_GE = _E // _BE


def _col128(row):
    rid = lax.broadcasted_iota(jnp.int32, (128, 128), 0)
    lid = lax.broadcasted_iota(jnp.int32, (128, 128), 1)
    b = jnp.broadcast_to(row, (128, 128))
    return jnp.sum(jnp.where(rid == lid, b, 0.0), axis=1, keepdims=True)


def _filter_body(attrs_ref, ew_ref, wf1T_ref, bf1_ref, wf2T_ref, w_ref):
    a = attrs_ref[...].astype(jnp.bfloat16)
    h = jnp.dot(a, wf1T_ref[...], preferred_element_type=jnp.float32)
    h = h + bf1_ref[...]
    h = h * jax.nn.sigmoid(h)
    w = jnp.dot(h.astype(jnp.bfloat16), wf2T_ref[...],
                preferred_element_type=jnp.float32)
    ew = ew_ref[0]
    c = 0.5 * (jnp.cos(jnp.pi * ew / _R_CUT) + 1.0)
    c = c * (ew < _R_CUT).astype(jnp.float32)
    ccol = jnp.concatenate(
        [_col128(c[r:r + 1, :]) for r in range(_BE // 128)], axis=0)
    w_ref[...] = w * ccol


def _edge_filter(edge_attrs, edge_weights, wf1T, bf1, wf2T):
    ew2 = edge_weights.reshape(_GE, _BE // 128, 128)
    return pl.pallas_call(
        _filter_body,
        grid=(_GE,),
        in_specs=[
            pl.BlockSpec((_BE, _D), lambda i: (i, 0)),
            pl.BlockSpec((1, _BE // 128, 128), lambda i: (i, 0, 0)),
            pl.BlockSpec((_D, _D), lambda i: (0, 0)),
            pl.BlockSpec((1, _D), lambda i: (0, 0)),
            pl.BlockSpec((_D, _D), lambda i: (0, 0)),
        ],
        out_specs=pl.BlockSpec((_BE, _D), lambda i: (i, 0)),
        out_shape=jax.ShapeDtypeStruct((_E, _D), jnp.float32),
    )(edge_attrs, ew2, wf1T, bf1.reshape(1, _D), wf2T)


_NC, _NS, _L = 2, 16, 16
_NW = _NC * _NS
_CH = 48
_PERW = _E // _NW
_NCH = 208
_TL = _PERW - _NCH * _CH
_NP = 10112
_RPT = _NP // _NS

_sc_mesh = plsc.VectorSubcoreMesh(core_axis_name="c", subcore_axis_name="s")


@functools.partial(
    pl.kernel,
    out_type=jax.ShapeDtypeStruct((_NC, _NP, _D), jnp.float32),
    mesh=_sc_mesh,
    scratch_types=[
        pltpu.VMEM((_PERW,), jnp.int32),
        pltpu.VMEM((_CH,), jnp.int32),
        pltpu.VMEM((_CH,), jnp.int32),
        pltpu.VMEM((_CH,), jnp.int32),
        pltpu.VMEM((_CH,), jnp.int32),
        pltpu.VMEM((_CH,), jnp.int32),
        pltpu.VMEM((_CH,), jnp.int32),
        pltpu.VMEM((_TL,), jnp.int32),
        pltpu.VMEM((_TL,), jnp.int32),
        pltpu.VMEM((_CH, _D), jnp.float32),
        pltpu.VMEM((_CH, _D), jnp.float32),
        pltpu.VMEM((_CH, _D), jnp.float32),
        pltpu.VMEM((_CH, _D), jnp.float32),
        pltpu.VMEM((_CH, _D), jnp.float32),
        pltpu.VMEM((_CH, _D), jnp.float32),
        pltpu.VMEM_SHARED((_NP, _D), jnp.float32),
        pltpu.SemaphoreType.DMA,
        pltpu.SemaphoreType.DMA,
        pltpu.SemaphoreType.DMA,
        pltpu.SemaphoreType.DMA,
        pltpu.SemaphoreType.DMA,
        pltpu.SemaphoreType.DMA,
        pltpu.SemaphoreType.DMA,
        pltpu.SemaphoreType.DMA,
        pltpu.SemaphoreType.DMA,
    ],
)
def _sc_scatter(w_hbm, s_hbm, pk_hbm, z_hbm, out_hbm,
                pk, srcb0, srcb1, srcb2, dstb0, dstb1, dstb2, srct, dstt,
                w0, w1, w2, v0, v1, v2, acc,
                semw0, semw1, semw2, semg0, semg1, semg2,
                sems0, sems1, sems2):
    cid = lax.axis_index("c")
    sid = lax.axis_index("s")
    wid = cid * _NS + sid
    pltpu.sync_copy(z_hbm, acc.at[pl.ds(sid * _RPT, _RPT)])
    pltpu.sync_copy(pk_hbm.at[wid], pk)
    plsc.subcore_barrier()

    mask = jnp.full((_L,), 0xFFFF, jnp.int32)

    def unpack_idx(j, srcb, dstb):
        for k in range(_CH // _L):
            x = pk[pl.ds(j * _CH + k * _L, _L)]
            srcb[pl.ds(k * _L, _L)] = x & mask
            dstb[pl.ds(k * _L, _L)] = lax.shift_right_logical(x, 16)

    def start_loads(j, w_v, v_v, srcb, dstb, semw, semg):
        unpack_idx(j, srcb, dstb)
        base = wid * _PERW + j * _CH
        pltpu.async_copy(w_hbm.at[pl.ds(base, _CH)], w_v, semw)
        pltpu.async_copy(s_hbm.at[srcb], v_v, semg)

    def wait_loads(j, w_v, v_v, srcb, semw, semg):
        base = wid * _PERW + j * _CH
        pltpu.make_async_copy(w_hbm.at[pl.ds(base, _CH)], w_v, semw).wait()
        pltpu.make_async_copy(s_hbm.at[srcb], v_v, semg).wait()

    def multiply(w_v, v_v, nrow):
        def row(r, c2):
            for c in range(_D // _L):
                sl = pl.ds(c * _L, _L)
                v_v[r, sl] = v_v[r, sl] * w_v[r, sl]
            return c2
        lax.fori_loop(0, nrow, row, 0)

    def start_scatter(v_v, dstb, sems):
        pltpu.async_copy(v_v, acc.at[dstb], sems, add=True)

    def wait_scatter(v_v, dstb, sems):
        pltpu.make_async_copy(v_v, acc.at[dstb], sems).wait()

    sets = (
        (srcb0, dstb0, w0, v0, semw0, semg0, sems0),
        (srcb1, dstb1, w1, v1, semw1, semg1, sems1),
        (srcb2, dstb2, w2, v2, semw2, semg2, sems2),
    )

    def process(j, cur, nxt):
        srcb, dstb, w_v, v_v, semw, semg, sems = cur
        srcbq, dstbq, w_q, v_q, semw_q, semg_q, sems_q = nxt

        @pl.when(jnp.logical_and(j >= 1, j + 2 < _NCH))
        def _():
            wait_scatter(v_q, dstbq, sems_q)

        @pl.when(j + 2 < _NCH)
        def _():
            start_loads(j + 2, w_q, v_q, srcbq, dstbq, semw_q, semg_q)

        wait_loads(j, w_v, v_v, srcb, semw, semg)
        multiply(w_v, v_v, _CH)
        start_scatter(v_v, dstb, sems)

    start_loads(0, w0, v0, srcb0, dstb0, semw0, semg0)
    start_loads(1, w1, v1, srcb1, dstb1, semw1, semg1)

    def body(j, carry):
        p = j % 3
        for i in range(3):
            @pl.when(p == i)
            def _(i=i):
                process(j, sets[i], sets[(i + 2) % 3])
        return carry

    lax.fori_loop(0, _NCH, body, 0)
    for c in (_NCH - 3, _NCH - 2, _NCH - 1):
        srcb, dstb, w_v, v_v, semw, semg, sems = sets[c % 3]
        wait_scatter(v_v, dstb, sems)

    xt = pk[pl.ds(_NCH * _CH, _TL)]
    srct[...] = xt & mask
    dstt[...] = lax.shift_right_logical(xt, 16)
    tbase = wid * _PERW + _NCH * _CH
    pltpu.sync_copy(w_hbm.at[pl.ds(tbase, _TL)], w0.at[pl.ds(0, _TL)])
    pltpu.async_copy(s_hbm.at[srct], v0.at[pl.ds(0, _TL)], semg0).wait()
    multiply(w0, v0, _TL)
    pltpu.sync_copy(v0.at[pl.ds(0, _TL)], acc.at[dstt], add=True)

    plsc.subcore_barrier()
    pltpu.sync_copy(acc.at[pl.ds(sid * _RPT, _RPT)],
                    out_hbm.at[cid, pl.ds(sid * _RPT, _RPT)])


_BN = 1000
_GN = _N // _BN


def _node_body(p_ref, recv_ref, w2T_ref, b2_ref, g_ref, be_ref, w3T_ref,
               b3_ref, o_ref):
    upd = p_ref[0] + p_ref[1]
    y = jnp.dot(upd, w2T_ref[...], preferred_element_type=jnp.float32)
    y = y + b2_ref[...]
    mu = jnp.mean(y, axis=-1, keepdims=True)
    yc = y - mu
    var = jnp.mean(yc * yc, axis=-1, keepdims=True)
    y = yc * lax.rsqrt(var + 1e-5) * g_ref[...] + be_ref[...]
    y = y * jax.nn.sigmoid(y)
    o_ref[...] = (jnp.dot(y, w3T_ref[...], preferred_element_type=jnp.float32)
                  + b3_ref[...] + recv_ref[...])


def _node_mlp(partials, recv, w2T, b2, gamma, beta, w3T, b3):
    return pl.pallas_call(
        _node_body,
        grid=(_GN,),
        in_specs=[
            pl.BlockSpec((_NC, _BN, _D), lambda i: (0, i, 0)),
            pl.BlockSpec((_BN, _D), lambda i: (i, 0)),
            pl.BlockSpec((_D, _D), lambda i: (0, 0)),
            pl.BlockSpec((1, _D), lambda i: (0, 0)),
            pl.BlockSpec((1, _D), lambda i: (0, 0)),
            pl.BlockSpec((1, _D), lambda i: (0, 0)),
            pl.BlockSpec((_D, _D), lambda i: (0, 0)),
            pl.BlockSpec((1, _D), lambda i: (0, 0)),
        ],
        out_specs=pl.BlockSpec((_BN, _D), lambda i: (i, 0)),
        out_shape=jax.ShapeDtypeStruct((_N, _D), jnp.float32),
    )(partials, recv, w2T, b2.reshape(1, _D), gamma.reshape(1, _D),
      beta.reshape(1, _D), w3T, b3.reshape(1, _D))


def kernel(senders, receivers, edge_indices, edge_weights, edge_versors,
           edge_attrs, W1, Wf1, bf1, Wf2, W2, b2, gamma, beta, W3, b3):
    del edge_versors
    s = _lin1(senders[0], W1.T)
    w = _edge_filter(edge_attrs, edge_weights,
                     Wf1.T.astype(jnp.bfloat16), bf1,
                     Wf2.T.astype(jnp.bfloat16))
    zeros = jnp.zeros((_RPT, _D), jnp.float32)
    packed = (edge_indices[0] | (edge_indices[1] << 16)).reshape(_NW, _PERW)
    partials = _sc_scatter(w, s, packed, zeros)
    return _node_mlp(partials, receivers[0], W2.T, b2, gamma, beta,
                     W3.T, b3)
